# Initial kernel scaffold; baseline (speedup 1.0000x reference)
import jax
import jax.numpy as jnp
from jax.experimental import pallas as pl


def kernel(x, query, keys, values, energy_score):
    raise NotImplementedError("write your pallas kernel here")


# speedup vs baseline: 1.2867x; 1.2867x over previous
import jax
import jax.numpy as jnp
from jax.experimental import pallas as pl
from jax.experimental.pallas import tpu as pltpu

_B = 4
_S = 2048
_D = 768
_CHUNK = 256


def _body(x_ref, q_ref, o_ref, accx, accq):
    i = pl.program_id(0)
    n = pl.num_programs(0)

    @pl.when(i == 0)
    def _init():
        accx[...] = jnp.zeros_like(accx)
        accq[...] = jnp.zeros_like(accq)

    accx[...] += jnp.sum(x_ref[...], axis=1)
    accq[...] += jnp.sum(q_ref[...], axis=1)

    @pl.when(i == n - 1)
    def _finish():
        xp = accx[...] * (1.0 / _S)
        qp = accq[...] * (1.0 / _S)
        attn = jax.lax.dot_general(
            qp, xp, (((1,), (1,)), ((), ())),
            preferred_element_type=jnp.float32)
        attn = jax.nn.softmax(attn, axis=-1)
        ctx = jnp.dot(attn, xp, preferred_element_type=jnp.float32)
        o_ref[...] = ctx[:, None, :]


def kernel(x, query, keys, values, energy_score):
    del keys, values, energy_score
    grid = (_S // _CHUNK,)
    return pl.pallas_call(
        _body,
        grid=grid,
        in_specs=[
            pl.BlockSpec((_B, _CHUNK, _D), lambda i: (0, i, 0)),
            pl.BlockSpec((_B, _CHUNK, _D), lambda i: (0, i, 0)),
        ],
        out_specs=pl.BlockSpec((_B, 1, _D), lambda i: (0, 0, 0)),
        out_shape=jax.ShapeDtypeStruct((_B, 1, _D), jnp.float32),
        scratch_shapes=[
            pltpu.VMEM((_B, _D), jnp.float32),
            pltpu.VMEM((_B, _D), jnp.float32),
        ],
    )(x, query)
